# Initial kernel scaffold; baseline (speedup 1.0000x reference)
import jax
import jax.numpy as jnp
from jax.experimental import pallas as pl


def kernel(symbols, positional_encoding):
    raise NotImplementedError("write your pallas kernel here")


# speedup vs baseline: 4.6665x; 4.6665x over previous
import jax
import jax.numpy as jnp
from jax.experimental import pallas as pl

B = 4
S = 8192
D = 768
S_BLK = 512


def _pe_kernel(sym_ref, pe_ref, out_ref):
    pe = pe_ref[0]
    mask = (sym_ref[...] != 0).astype(jnp.float32)
    out_ref[...] = pe[None, :, :] * mask[:, :, None]


def kernel(symbols, positional_encoding):
    grid = (S // S_BLK,)
    return pl.pallas_call(
        _pe_kernel,
        grid=grid,
        in_specs=[
            pl.BlockSpec((B, S_BLK), lambda i: (0, i)),
            pl.BlockSpec((1, S_BLK, D), lambda i: (0, i, 0)),
        ],
        out_specs=pl.BlockSpec((B, S_BLK, D), lambda i: (0, i, 0)),
        out_shape=jax.ShapeDtypeStruct((B, S, D), jnp.float32),
    )(symbols, positional_encoding)


# speedup vs baseline: 4.8502x; 1.0394x over previous
import jax
import jax.numpy as jnp
from jax.experimental import pallas as pl

B = 4
S = 8192
D = 768
S_BLK = 1024


def _pe_kernel(sym_ref, pe_ref, out_ref):
    pe = pe_ref[0]
    mask = (sym_ref[...] != 0).astype(jnp.float32)
    out_ref[...] = pe[None, :, :] * mask[:, :, None]


def kernel(symbols, positional_encoding):
    grid = (S // S_BLK,)
    return pl.pallas_call(
        _pe_kernel,
        grid=grid,
        in_specs=[
            pl.BlockSpec((B, S_BLK), lambda i: (0, i)),
            pl.BlockSpec((1, S_BLK, D), lambda i: (0, i, 0)),
        ],
        out_specs=pl.BlockSpec((B, S_BLK, D), lambda i: (0, i, 0)),
        out_shape=jax.ShapeDtypeStruct((B, S, D), jnp.float32),
    )(symbols, positional_encoding)


# speedup vs baseline: 4.8549x; 1.0010x over previous
import jax
import jax.numpy as jnp
from jax.experimental import pallas as pl
from jax.experimental.pallas import tpu as pltpu

B = 4
S = 8192
D = 768
S_BLK = 1024


def _pe_kernel(sym_ref, pe_ref, out_ref):
    pe = pe_ref[0]
    mask = (sym_ref[...] != 0).astype(jnp.float32)
    out_ref[...] = pe[None, :, :] * mask[:, :, None]


def kernel(symbols, positional_encoding):
    grid = (S // S_BLK,)
    return pl.pallas_call(
        _pe_kernel,
        grid=grid,
        in_specs=[
            pl.BlockSpec((B, S_BLK), lambda i: (0, i)),
            pl.BlockSpec((1, S_BLK, D), lambda i: (0, i, 0)),
        ],
        out_specs=pl.BlockSpec((B, S_BLK, D), lambda i: (0, i, 0)),
        out_shape=jax.ShapeDtypeStruct((B, S, D), jnp.float32),
        compiler_params=pltpu.CompilerParams(
            dimension_semantics=("parallel",),
        ),
    )(symbols, positional_encoding)
